# baseline (speedup 1.0000x reference)
import functools

import jax
import jax.numpy as jnp
from jax import lax
from jax.experimental import pallas as pl
from jax.experimental.pallas import tpu as pltpu
from jax.experimental.pallas import tpu_sc as plsc

_B = 16384
_D = 64
_DP = 128
_L = 50
_LP = 52
_NG = _LP // 4
_TB = 256
_CH = 128
_NW = 32


_NB = 2
_CH2 = 256
_NV = _CH2 // 16


def _sc_gather(news_pairs, user_pairs, hist_idx, news_idx, user_idx):
    bh = hist_idx.shape[0]
    n_h = bh // (_NW * _CH2)
    n_b = _B // (_NW * _CH2)
    mesh = plsc.VectorSubcoreMesh(core_axis_name="c", subcore_axis_name="s")
    out_types = (
        jax.ShapeDtypeStruct((bh, _DP), jnp.float32),
        jax.ShapeDtypeStruct((_B, _DP), jnp.float32),
        jax.ShapeDtypeStruct((_B, _DP), jnp.float32),
    )
    scratch = (
        [pltpu.VMEM((n_h * _CH2,), jnp.int32),
         pltpu.VMEM((n_b * _CH2,), jnp.int32),
         pltpu.VMEM((n_b * _CH2,), jnp.int32)]
        + [pltpu.VMEM((_CH2, _DP), jnp.float32) for _ in range(_NB)]
        + [pltpu.SemaphoreType.DMA for _ in range(2 * _NB)]
    )

    @functools.partial(pl.kernel, mesh=mesh, out_type=out_types,
                       scratch_types=scratch)
    def k(news_t, user_t, hidx, nidx, uidx, out_h, out_n, out_u,
          hidx_v, nidx_v, uidx_v, *bufs_sems):
        bufs = bufs_sems[:_NB]
        gsem = bufs_sems[_NB:2 * _NB]
        wsem = bufs_sems[2 * _NB:]
        wid = lax.axis_index("s") * 2 + lax.axis_index("c")

        def pipe(table, idx_v, out_hbm, nchunks):
            base = wid * nchunks * _CH2

            def g_start(b, i):
                for j in range(_NV):
                    vec = idx_v[pl.ds(i * _CH2 + j * 16, 16)]
                    pltpu.make_async_copy(
                        table.at[vec], bufs[b].at[pl.ds(j * 16, 16)],
                        gsem[b]).start()

            def g_drain(b):
                pltpu.make_async_copy(
                    table.at[pl.ds(0, _CH2)], bufs[b], gsem[b]).wait()

            def w_copy(b, i):
                dst = out_hbm.at[pl.ds(base + i * _CH2, _CH2)]
                return pltpu.make_async_copy(bufs[b], dst, wsem[b])

            for b in range(_NB):
                g_start(b, b)
            ng = nchunks // _NB

            @pl.loop(0, ng - 1)
            def _(g):
                for b in range(_NB):
                    g_drain(b)
                    w_copy(b, g * _NB + b).start()
                for b in range(_NB):
                    w_copy(b, g * _NB + b).wait()
                    g_start(b, (g + 1) * _NB + b)

            for b in range(_NB):
                g_drain(b)
                w_copy(b, (ng - 1) * _NB + b).start()
            for b in range(_NB):
                w_copy(b, (ng - 1) * _NB + b).wait()

        pltpu.sync_copy(hidx.at[pl.ds(wid * n_h * _CH2, n_h * _CH2)], hidx_v)
        pltpu.sync_copy(nidx.at[pl.ds(wid * n_b * _CH2, n_b * _CH2)], nidx_v)
        pltpu.sync_copy(uidx.at[pl.ds(wid * n_b * _CH2, n_b * _CH2)], uidx_v)
        pipe(news_t, hidx_v, out_h, n_h)
        pipe(news_t, nidx_v, out_n, n_b)
        pipe(user_t, uidx_v, out_u, n_b)

    return k(news_pairs, user_pairs, hist_idx, news_idx, user_idx)


def _half(pair, idx_col):
    odd = (idx_col & 1) == 1
    return jnp.where(odd, pair[:, _D:], pair[:, :_D])


def _tc_body(hist_ref, hidx_ref, upair_ref, uid_ref, npair_ref, nid_ref,
             w1s_ref, b1s_ref, w2_ref, b2_ref, wu_ref, bu_ref, wn_ref,
             bn_ref, out_ref):
    w2 = w2_ref[...]
    b2 = b2_ref[...]
    hidx = hidx_ref[...]
    neg = jnp.float32(-1e9)
    m = jnp.full((_TB, 1), -1e30, jnp.float32)
    s = jnp.zeros((_TB, 1), jnp.float32)
    acc = jnp.zeros((_TB, _D), jnp.float32)
    for g in range(_NG):
        xs = []
        for kk in range(4):
            l = 4 * g + kk
            pair = hist_ref[:, l * _DP:(l + 1) * _DP]
            xs.append(_half(pair, hidx[:, l:l + 1]))
        x4 = jnp.concatenate(xs, axis=1)
        h4 = jnp.tanh(
            jnp.dot(x4, w1s_ref[...], preferred_element_type=jnp.float32)
            + b1s_ref[...])
        for kk in range(4):
            l = 4 * g + kk
            if l >= _L:
                continue
            a = jnp.sum(h4[:, kk * 32:(kk + 1) * 32] * w2, axis=1,
                        keepdims=True) + b2
            a = jnp.where(hidx[:, l:l + 1] != 0, a, neg)
            m2 = jnp.maximum(m, a)
            c = jnp.exp(m - m2)
            p = jnp.exp(a - m2)
            s = s * c + p
            acc = acc * c + p * xs[kk]
            m = m2
    hist_repr = acc / s
    uemb = _half(upair_ref[...], uid_ref[...])
    nemb = _half(npair_ref[...], nid_ref[...])
    u = uemb + hist_repr
    ur = jnp.maximum(
        jnp.dot(u, wu_ref[...], preferred_element_type=jnp.float32)
        + bu_ref[...], 0.0)
    nr = jnp.maximum(
        jnp.dot(nemb, wn_ref[...], preferred_element_type=jnp.float32)
        + bn_ref[...], 0.0)
    out_ref[...] = jax.nn.sigmoid(jnp.sum(ur * nr, axis=1, keepdims=True))


def _tc_call(hist2d, history_p, gath_u, user_idx, gath_n, news_idx,
             w1s, b1s, w2r, b2r, W_user, b_user, W_news, b_news):
    grid = _B // _TB
    return pl.pallas_call(
        _tc_body,
        grid=(grid,),
        in_specs=[
            pl.BlockSpec((_TB, _LP * _DP), lambda i: (i, 0)),
            pl.BlockSpec((_TB, _LP), lambda i: (i, 0)),
            pl.BlockSpec((_TB, _DP), lambda i: (i, 0)),
            pl.BlockSpec((_TB, 1), lambda i: (i, 0)),
            pl.BlockSpec((_TB, _DP), lambda i: (i, 0)),
            pl.BlockSpec((_TB, 1), lambda i: (i, 0)),
            pl.BlockSpec((4 * _D, 128), lambda i: (0, 0)),
            pl.BlockSpec((1, 128), lambda i: (0, 0)),
            pl.BlockSpec((1, 32), lambda i: (0, 0)),
            pl.BlockSpec((1, 1), lambda i: (0, 0)),
            pl.BlockSpec((_D, _D), lambda i: (0, 0)),
            pl.BlockSpec((1, _D), lambda i: (0, 0)),
            pl.BlockSpec((_D, _D), lambda i: (0, 0)),
            pl.BlockSpec((1, _D), lambda i: (0, 0)),
        ],
        out_specs=pl.BlockSpec((_TB, 1), lambda i: (i, 0)),
        out_shape=jax.ShapeDtypeStruct((_B, 1), jnp.float32),
    )(hist2d, history_p, gath_u, user_idx.reshape(_B, 1),
      gath_n, news_idx.reshape(_B, 1), w1s, b1s, w2r, b2r,
      W_user, b_user[None, :], W_news, b_news[None, :])


def kernel(user_idx, news_idx, history, user_table, news_table, W_user,
           b_user, W_news, b_news, W_a1, b_a1, W_a2, b_a2):
    history_p = jnp.concatenate(
        [history, jnp.zeros((_B, _LP - _L), history.dtype)], axis=1)
    hist_pair_idx = (history_p >> 1).reshape(-1)

    news_pairs = news_table.reshape(news_table.shape[0] // 2, _DP)
    user_pairs = user_table.reshape(user_table.shape[0] // 2, _DP)

    gath_h, gath_n, gath_u = _sc_gather(
        news_pairs, user_pairs, hist_pair_idx, news_idx >> 1, user_idx >> 1)
    hist2d = gath_h.reshape(_B, _LP * _DP)

    w1s = jnp.kron(jnp.eye(4, dtype=jnp.float32), W_a1)
    b1s = jnp.tile(b_a1, 4)[None, :]
    w2r = W_a2[:, 0][None, :]
    b2r = b_a2.reshape(1, 1)

    out = _tc_call(hist2d, history_p, gath_u, user_idx, gath_n, news_idx,
                   w1s, b1s, w2r, b2r, W_user, b_user, W_news, b_news)
    return out[:, 0]


# speedup vs baseline: 1.7166x; 1.7166x over previous
import functools

import jax
import jax.numpy as jnp
from jax import lax
from jax.experimental import pallas as pl
from jax.experimental.pallas import tpu as pltpu
from jax.experimental.pallas import tpu_sc as plsc

_B = 16384
_D = 64
_DP = 128
_L = 50
_LP = 52
_TB = 256
_NW = 32
_NB = 2
_CH2 = 256
_NV = _CH2 // 16


def _sc_gather(news_pairs, user_pairs, hist_idx, news_idx, user_idx):
    bh = hist_idx.shape[0]
    n_h = bh // (_NW * _CH2)
    n_b = _B // (_NW * _CH2)
    mesh = plsc.VectorSubcoreMesh(core_axis_name="c", subcore_axis_name="s")
    out_types = (
        jax.ShapeDtypeStruct((bh, _DP), jnp.float32),
        jax.ShapeDtypeStruct((_B, _DP), jnp.float32),
        jax.ShapeDtypeStruct((_B, _DP), jnp.float32),
    )
    scratch = (
        [pltpu.VMEM((n_h * _CH2,), jnp.int32),
         pltpu.VMEM((n_b * _CH2,), jnp.int32),
         pltpu.VMEM((n_b * _CH2,), jnp.int32)]
        + [pltpu.VMEM((_CH2, _DP), jnp.float32) for _ in range(_NB)]
        + [pltpu.SemaphoreType.DMA for _ in range(2 * _NB)]
    )

    @functools.partial(pl.kernel, mesh=mesh, out_type=out_types,
                       scratch_types=scratch)
    def k(news_t, user_t, hidx, nidx, uidx, out_h, out_n, out_u,
          hidx_v, nidx_v, uidx_v, *bufs_sems):
        bufs = bufs_sems[:_NB]
        gsem = bufs_sems[_NB:2 * _NB]
        wsem = bufs_sems[2 * _NB:]
        wid = lax.axis_index("s") * 2 + lax.axis_index("c")

        def pipe(table, idx_v, out_hbm, nchunks):
            base = wid * nchunks * _CH2

            def g_start(b, i):
                for j in range(_NV):
                    vec = idx_v[pl.ds(i * _CH2 + j * 16, 16)]
                    pltpu.make_async_copy(
                        table.at[vec], bufs[b].at[pl.ds(j * 16, 16)],
                        gsem[b]).start()

            def g_drain(b):
                pltpu.make_async_copy(
                    table.at[pl.ds(0, _CH2)], bufs[b], gsem[b]).wait()

            def w_copy(b, i):
                dst = out_hbm.at[pl.ds(base + i * _CH2, _CH2)]
                return pltpu.make_async_copy(bufs[b], dst, wsem[b])

            for b in range(_NB):
                g_start(b, b)
            ng = nchunks // _NB

            @pl.loop(0, ng - 1)
            def _(g):
                for b in range(_NB):
                    g_drain(b)
                    w_copy(b, g * _NB + b).start()
                for b in range(_NB):
                    w_copy(b, g * _NB + b).wait()
                    g_start(b, (g + 1) * _NB + b)

            for b in range(_NB):
                g_drain(b)
                w_copy(b, (ng - 1) * _NB + b).start()
            for b in range(_NB):
                w_copy(b, (ng - 1) * _NB + b).wait()

        pltpu.sync_copy(hidx.at[pl.ds(wid * n_h * _CH2, n_h * _CH2)], hidx_v)
        pltpu.sync_copy(nidx.at[pl.ds(wid * n_b * _CH2, n_b * _CH2)], nidx_v)
        pltpu.sync_copy(uidx.at[pl.ds(wid * n_b * _CH2, n_b * _CH2)], uidx_v)
        pipe(news_t, hidx_v, out_h, n_h)
        pipe(news_t, nidx_v, out_n, n_b)
        pipe(user_t, uidx_v, out_u, n_b)

    return k(news_pairs, user_pairs, hist_idx, news_idx, user_idx)


def _half(pair, idx_col):
    odd = (idx_col & 1) == 1
    return jnp.where(odd, pair[:, _D:], pair[:, :_D])


def _tc_body(hist_ref, hidx_ref, msk_ref, negc_ref, upair_ref, uid_ref,
             npair_ref, nid_ref, w1p_ref, b1p_ref, w22_ref, pfold_ref,
             pdup_ref, r2_ref, b2_ref, wu_ref, bu_ref, wn_ref, bn_ref,
             out_ref):
    msk = msk_ref[...]
    a_parts = []
    for l in range(_LP):
        x = hist_ref[l]
        h = jnp.tanh(
            jnp.dot(x, w1p_ref[...], preferred_element_type=jnp.float32)
            + b1p_ref[...])
        a_parts.append(
            jnp.dot(h, w22_ref[...], preferred_element_type=jnp.float32))
    a104 = jnp.concatenate(a_parts, axis=1)
    a52 = jnp.dot(a104 * msk, pfold_ref[...],
                  preferred_element_type=jnp.float32) + b2_ref[...]
    a52 = jnp.where(hidx_ref[...] != 0, a52, negc_ref[...])
    m = jnp.max(a52, axis=1, keepdims=True)
    e = jnp.exp(a52 - m)
    s = jnp.sum(e, axis=1, keepdims=True)
    w52 = e / s
    w104 = jnp.dot(w52, pdup_ref[...],
                   preferred_element_type=jnp.float32)
    wexp = jnp.dot(w104 * msk, r2_ref[...],
                   preferred_element_type=jnp.float32)
    acc = jnp.zeros((_TB, _DP), jnp.float32)
    for l in range(_LP):
        acc = acc + hist_ref[l] * wexp[:, l * _DP:(l + 1) * _DP]
    hist_repr = acc[:, :_D] + acc[:, _D:]
    uemb = _half(upair_ref[...], uid_ref[...])
    nemb = _half(npair_ref[...], nid_ref[...])
    u = uemb + hist_repr
    ur = jnp.maximum(
        jnp.dot(u, wu_ref[...], preferred_element_type=jnp.float32)
        + bu_ref[...], 0.0)
    nr = jnp.maximum(
        jnp.dot(nemb, wn_ref[...], preferred_element_type=jnp.float32)
        + bn_ref[...], 0.0)
    out_ref[...] = jax.nn.sigmoid(jnp.sum(ur * nr, axis=1, keepdims=True))


def _tc_call(hist3, history_p, msk104, negc, gath_u, user_idx, gath_n,
             news_idx, w1p, b1p, w22, pfold, pdup, r2, b2r,
             W_user, b_user, W_news, b_news):
    grid = _B // _TB
    return pl.pallas_call(
        _tc_body,
        grid=(grid,),
        in_specs=[
            pl.BlockSpec((_LP, _TB, _DP), lambda i: (0, i, 0)),
            pl.BlockSpec((_TB, _LP), lambda i: (i, 0)),
            pl.BlockSpec((_TB, 2 * _LP), lambda i: (i, 0)),
            pl.BlockSpec((1, _LP), lambda i: (0, 0)),
            pl.BlockSpec((_TB, _DP), lambda i: (i, 0)),
            pl.BlockSpec((_TB, 1), lambda i: (i, 0)),
            pl.BlockSpec((_TB, _DP), lambda i: (i, 0)),
            pl.BlockSpec((_TB, 1), lambda i: (i, 0)),
            pl.BlockSpec((_DP, _D), lambda i: (0, 0)),
            pl.BlockSpec((1, _D), lambda i: (0, 0)),
            pl.BlockSpec((_D, 2), lambda i: (0, 0)),
            pl.BlockSpec((2 * _LP, _LP), lambda i: (0, 0)),
            pl.BlockSpec((_LP, 2 * _LP), lambda i: (0, 0)),
            pl.BlockSpec((2 * _LP, _LP * _DP), lambda i: (0, 0)),
            pl.BlockSpec((1, 1), lambda i: (0, 0)),
            pl.BlockSpec((_D, _D), lambda i: (0, 0)),
            pl.BlockSpec((1, _D), lambda i: (0, 0)),
            pl.BlockSpec((_D, _D), lambda i: (0, 0)),
            pl.BlockSpec((1, _D), lambda i: (0, 0)),
        ],
        out_specs=pl.BlockSpec((_TB, 1), lambda i: (i, 0)),
        out_shape=jax.ShapeDtypeStruct((_B, 1), jnp.float32),
    )(hist3, history_p, msk104, negc, gath_u, user_idx.reshape(_B, 1),
      gath_n, news_idx.reshape(_B, 1), w1p, b1p, w22, pfold, pdup, r2,
      b2r, W_user, b_user[None, :], W_news, b_news[None, :])


def kernel(user_idx, news_idx, history, user_table, news_table, W_user,
           b_user, W_news, b_news, W_a1, b_a1, W_a2, b_a2):
    history_p = jnp.concatenate(
        [history, jnp.zeros((_B, _LP - _L), history.dtype)], axis=1)
    hist_pair_idx = (history_p >> 1).T.reshape(-1)

    news_pairs = news_table.reshape(news_table.shape[0] // 2, _DP)
    user_pairs = user_table.reshape(user_table.shape[0] // 2, _DP)

    gath_h, gath_n, gath_u = _sc_gather(
        news_pairs, user_pairs, hist_pair_idx, news_idx >> 1, user_idx >> 1)
    hist3 = gath_h.reshape(_LP, _B, _DP)

    par = (history_p & 1).astype(jnp.float32)
    msk104 = jnp.stack([1.0 - par, par], axis=-1).reshape(_B, 2 * _LP)
    negc = jnp.concatenate(
        [jnp.full((1, _L), -1e9, jnp.float32),
         jnp.full((1, _LP - _L), -jnp.inf, jnp.float32)], axis=1)

    eye2 = jnp.eye(2, dtype=jnp.float32)
    w1p = jnp.kron(eye2, W_a1)
    b1p = jnp.tile(b_a1, 2)[None, :]
    w22 = jnp.kron(eye2, W_a2)
    eye52 = jnp.eye(_LP, dtype=jnp.float32)
    pfold = jnp.kron(eye52, jnp.ones((2, 1), jnp.float32))
    pdup = jnp.kron(eye52, jnp.ones((1, 2), jnp.float32))
    sel = jnp.zeros((2, _DP), jnp.float32)
    sel = sel.at[0, :_D].set(1.0).at[1, _D:].set(1.0)
    r2 = jnp.kron(eye52, sel)
    b2r = b_a2.reshape(1, 1)

    out = _tc_call(hist3, history_p, msk104, negc, gath_u, user_idx,
                   gath_n, news_idx, w1p, b1p, w22, pfold, pdup, r2, b2r,
                   W_user, b_user, W_news, b_news)
    return out[:, 0]


# speedup vs baseline: 1.8670x; 1.0877x over previous
import functools

import jax
import jax.numpy as jnp
from jax import lax
from jax.experimental import pallas as pl
from jax.experimental.pallas import tpu as pltpu
from jax.experimental.pallas import tpu_sc as plsc

_B = 16384
_D = 64
_DP = 128
_L = 50
_LP = 52
_TB = 256
_NW = 32
_NB = 2
_CH2 = 256
_NV = _CH2 // 16


def _sc_pipe(wid, table, idx_v, out_hbm, nchunks, bufs, gsem, wsem,
             interleaved):
    def obase(i):
        if interleaved:
            return (i * _NW + wid) * _CH2
        return (wid * nchunks + i) * _CH2

    def g_start(b, i):
        for j in range(_NV):
            vec = idx_v[pl.ds(i * _CH2 + j * 16, 16)]
            pltpu.make_async_copy(
                table.at[vec], bufs[b].at[pl.ds(j * 16, 16)],
                gsem[b]).start()

    def g_drain(b):
        for j in range(_NV):
            pltpu.make_async_copy(
                table.at[pl.ds(0, 16)],
                bufs[b].at[pl.ds(j * 16, 16)], gsem[b]).wait()

    def w_copy(b, i):
        dst = out_hbm.at[pl.ds(obase(i), _CH2)]
        return pltpu.make_async_copy(bufs[b], dst, wsem[b])

    for b in range(_NB):
        g_start(b, b)
    ng = nchunks // _NB

    @pl.loop(0, ng - 1)
    def _(g):
        for b in range(_NB):
            g_drain(b)
            w_copy(b, g * _NB + b).start()
        for b in range(_NB):
            w_copy(b, g * _NB + b).wait()
            g_start(b, (g + 1) * _NB + b)

    for b in range(_NB):
        g_drain(b)
        w_copy(b, (ng - 1) * _NB + b).start()
    for b in range(_NB):
        w_copy(b, (ng - 1) * _NB + b).wait()


def _sc_gather_news(news_pairs, hist_idx_w, news_idx):
    bh = hist_idx_w.shape[0]
    n_h = bh // (_NW * _CH2)
    n_b = _B // (_NW * _CH2)
    mesh = plsc.VectorSubcoreMesh(core_axis_name="c", subcore_axis_name="s")
    out_types = (
        jax.ShapeDtypeStruct((bh, _DP), jnp.float32),
        jax.ShapeDtypeStruct((_B, _DP), jnp.float32),
    )
    scratch = (
        [pltpu.VMEM((n_h * _CH2,), jnp.int32),
         pltpu.VMEM((n_b * _CH2,), jnp.int32)]
        + [pltpu.VMEM((_CH2, _DP), jnp.float32) for _ in range(_NB)]
        + [pltpu.SemaphoreType.DMA for _ in range(2 * _NB)]
    )

    @functools.partial(pl.kernel, mesh=mesh, out_type=out_types,
                       scratch_types=scratch)
    def k(news_t, hidx, nidx, out_h, out_n, hidx_v, nidx_v, *bufs_sems):
        bufs = bufs_sems[:_NB]
        gsem = bufs_sems[_NB:2 * _NB]
        wsem = bufs_sems[2 * _NB:]
        wid = lax.axis_index("s") * 2 + lax.axis_index("c")
        pltpu.sync_copy(hidx.at[pl.ds(wid * n_h * _CH2, n_h * _CH2)], hidx_v)
        pltpu.sync_copy(nidx.at[pl.ds(wid * n_b * _CH2, n_b * _CH2)], nidx_v)
        _sc_pipe(wid, news_t, hidx_v, out_h, n_h, bufs, gsem, wsem, True)
        _sc_pipe(wid, news_t, nidx_v, out_n, n_b, bufs, gsem, wsem, False)

    return k(news_pairs, hist_idx_w, news_idx)


def _sc_gather_user(user_pairs, user_idx):
    n_b = _B // (_NW * _CH2)
    mesh = plsc.VectorSubcoreMesh(core_axis_name="c", subcore_axis_name="s")
    scratch = (
        [pltpu.VMEM((n_b * _CH2,), jnp.int32)]
        + [pltpu.VMEM((_CH2, _DP), jnp.float32) for _ in range(_NB)]
        + [pltpu.SemaphoreType.DMA for _ in range(2 * _NB)]
    )

    @functools.partial(
        pl.kernel, mesh=mesh,
        out_type=jax.ShapeDtypeStruct((_B, _DP), jnp.float32),
        scratch_types=scratch)
    def k(user_t, uidx, out_u, uidx_v, *bufs_sems):
        bufs = bufs_sems[:_NB]
        gsem = bufs_sems[_NB:2 * _NB]
        wsem = bufs_sems[2 * _NB:]
        wid = lax.axis_index("s") * 2 + lax.axis_index("c")
        pltpu.sync_copy(uidx.at[pl.ds(wid * n_b * _CH2, n_b * _CH2)], uidx_v)
        _sc_pipe(wid, user_t, uidx_v, out_u, n_b, bufs, gsem, wsem, False)

    return k(user_pairs, user_idx)


def _half(pair, idx_col):
    odd = (idx_col & 1) == 1
    return jnp.where(odd, pair[:, _D:], pair[:, :_D])


def _tc_body(hist_ref, hidx_ref, msk_ref, negc_ref, upair_ref, uid_ref,
             npair_ref, nid_ref, w1p_ref, b1p_ref, w22_ref, pfold_ref,
             pdup_ref, r2_ref, b2_ref, wu_ref, bu_ref, wn_ref, bn_ref,
             out_ref):
    msk = msk_ref[...]
    a_parts = []
    for l in range(_LP):
        x = hist_ref[l].astype(jnp.float32)
        h = jnp.tanh(
            jnp.dot(x, w1p_ref[...], preferred_element_type=jnp.float32)
            + b1p_ref[...])
        a_parts.append(
            jnp.dot(h, w22_ref[...], preferred_element_type=jnp.float32))
    a104 = jnp.concatenate(a_parts, axis=1)
    a52 = jnp.dot(a104 * msk, pfold_ref[...],
                  preferred_element_type=jnp.float32) + b2_ref[...]
    a52 = jnp.where(hidx_ref[...] != 0, a52, negc_ref[...])
    m = jnp.max(a52, axis=1, keepdims=True)
    e = jnp.exp(a52 - m)
    s = jnp.sum(e, axis=1, keepdims=True)
    w52 = e / s
    w104 = jnp.dot(w52, pdup_ref[...],
                   preferred_element_type=jnp.float32)
    wexp = jnp.dot(w104 * msk, r2_ref[...],
                   preferred_element_type=jnp.float32)
    acc = jnp.zeros((_TB, _DP), jnp.float32)
    for l in range(_LP):
        acc = acc + hist_ref[l].astype(jnp.float32) * wexp[:, l * _DP:(l + 1) * _DP]
    hist_repr = acc[:, :_D] + acc[:, _D:]
    uemb = _half(upair_ref[...].astype(jnp.float32), uid_ref[...])
    nemb = _half(npair_ref[...].astype(jnp.float32), nid_ref[...])
    u = uemb + hist_repr
    ur = jnp.maximum(
        jnp.dot(u, wu_ref[...], preferred_element_type=jnp.float32)
        + bu_ref[...], 0.0)
    nr = jnp.maximum(
        jnp.dot(nemb, wn_ref[...], preferred_element_type=jnp.float32)
        + bn_ref[...], 0.0)
    out_ref[...] = jax.nn.sigmoid(jnp.sum(ur * nr, axis=1, keepdims=True))


def _tc_call(hist3, history_p, msk104, negc, gath_u, user_idx, gath_n,
             news_idx, w1p, b1p, w22, pfold, pdup, r2, b2r,
             W_user, b_user, W_news, b_news):
    grid = _B // _TB
    return pl.pallas_call(
        _tc_body,
        grid=(grid,),
        in_specs=[
            pl.BlockSpec((_LP, _TB, _DP), lambda i: (0, i, 0)),
            pl.BlockSpec((_TB, _LP), lambda i: (i, 0)),
            pl.BlockSpec((_TB, 2 * _LP), lambda i: (i, 0)),
            pl.BlockSpec((1, _LP), lambda i: (0, 0)),
            pl.BlockSpec((_TB, _DP), lambda i: (i, 0)),
            pl.BlockSpec((_TB, 1), lambda i: (i, 0)),
            pl.BlockSpec((_TB, _DP), lambda i: (i, 0)),
            pl.BlockSpec((_TB, 1), lambda i: (i, 0)),
            pl.BlockSpec((_DP, _D), lambda i: (0, 0)),
            pl.BlockSpec((1, _D), lambda i: (0, 0)),
            pl.BlockSpec((_D, 2), lambda i: (0, 0)),
            pl.BlockSpec((2 * _LP, _LP), lambda i: (0, 0)),
            pl.BlockSpec((_LP, 2 * _LP), lambda i: (0, 0)),
            pl.BlockSpec((2 * _LP, _LP * _DP), lambda i: (0, 0)),
            pl.BlockSpec((1, 1), lambda i: (0, 0)),
            pl.BlockSpec((_D, _D), lambda i: (0, 0)),
            pl.BlockSpec((1, _D), lambda i: (0, 0)),
            pl.BlockSpec((_D, _D), lambda i: (0, 0)),
            pl.BlockSpec((1, _D), lambda i: (0, 0)),
        ],
        out_specs=pl.BlockSpec((_TB, 1), lambda i: (i, 0)),
        out_shape=jax.ShapeDtypeStruct((_B, 1), jnp.float32),
    )(hist3, history_p, msk104, negc, gath_u, user_idx.reshape(_B, 1),
      gath_n, news_idx.reshape(_B, 1), w1p, b1p, w22, pfold, pdup, r2,
      b2r, W_user, b_user[None, :], W_news, b_news[None, :])


def kernel(user_idx, news_idx, history, user_table, news_table, W_user,
           b_user, W_news, b_news, W_a1, b_a1, W_a2, b_a2):
    history_p = jnp.concatenate(
        [history, jnp.zeros((_B, _LP - _L), history.dtype)], axis=1)
    hist_pair_idx = (history_p >> 1).T.reshape(-1)

    news_pairs = news_table.reshape(news_table.shape[0] // 2, _DP)
    user_pairs = user_table.reshape(user_table.shape[0] // 2, _DP)

    nch = hist_pair_idx.shape[0] // _CH2
    hist_idx_w = (hist_pair_idx.reshape(nch // _NW, _NW, _CH2)
                  .transpose(1, 0, 2).reshape(-1))

    gath_h, gath_n = _sc_gather_news(news_pairs, hist_idx_w, news_idx >> 1)
    gath_u = _sc_gather_user(user_pairs, user_idx >> 1)
    hist3 = gath_h.reshape(_LP, _B, _DP)

    par = (history_p & 1).astype(jnp.float32)
    msk104 = jnp.stack([1.0 - par, par], axis=-1).reshape(_B, 2 * _LP)
    negc = jnp.concatenate(
        [jnp.full((1, _L), -1e9, jnp.float32),
         jnp.full((1, _LP - _L), -jnp.inf, jnp.float32)], axis=1)

    eye2 = jnp.eye(2, dtype=jnp.float32)
    w1p = jnp.kron(eye2, W_a1)
    b1p = jnp.tile(b_a1, 2)[None, :]
    w22 = jnp.kron(eye2, W_a2)
    eye52 = jnp.eye(_LP, dtype=jnp.float32)
    pfold = jnp.kron(eye52, jnp.ones((2, 1), jnp.float32))
    pdup = jnp.kron(eye52, jnp.ones((1, 2), jnp.float32))
    sel = jnp.zeros((2, _DP), jnp.float32)
    sel = sel.at[0, :_D].set(1.0).at[1, _D:].set(1.0)
    r2 = jnp.kron(eye52, sel)
    b2r = b_a2.reshape(1, 1)

    out = _tc_call(hist3, history_p, msk104, negc, gath_u, user_idx,
                   gath_n, news_idx, w1p, b1p, w22, pfold, pdup, r2, b2r,
                   W_user, b_user, W_news, b_news)
    return out[:, 0]


# speedup vs baseline: 1.9328x; 1.0352x over previous
import functools

import jax
import jax.numpy as jnp
from jax import lax
from jax.experimental import pallas as pl
from jax.experimental.pallas import tpu as pltpu
from jax.experimental.pallas import tpu_sc as plsc

_B = 16384
_D = 64
_DP = 128
_L = 50
_LP = 52
_TB = 256
_NW = 32
_NB = 2
_CH2 = 256
_NV = _CH2 // 16


def _sc_pipe(wid, table, idx_v, out_hbm, nchunks, bufs, gsem, wsem,
             interleaved):
    def obase(i):
        if interleaved:
            return (i * _NW + wid) * _CH2
        return (wid * nchunks + i) * _CH2

    def g_start(b, i):
        for j in range(_NV):
            vec = idx_v[pl.ds(i * _CH2 + j * 16, 16)]
            pltpu.make_async_copy(
                table.at[vec], bufs[b].at[pl.ds(j * 16, 16)],
                gsem[b]).start()

    def g_drain(b):
        for j in range(_NV):
            pltpu.make_async_copy(
                table.at[pl.ds(0, 16)],
                bufs[b].at[pl.ds(j * 16, 16)], gsem[b]).wait()

    def w_copy(b, i):
        dst = out_hbm.at[pl.ds(obase(i), _CH2)]
        return pltpu.make_async_copy(bufs[b], dst, wsem[b])

    for b in range(_NB):
        g_start(b, b)
    ng = nchunks // _NB

    @pl.loop(0, ng - 1)
    def _(g):
        for b in range(_NB):
            g_drain(b)
            w_copy(b, g * _NB + b).start()
        for b in range(_NB):
            w_copy(b, g * _NB + b).wait()
            g_start(b, (g + 1) * _NB + b)

    for b in range(_NB):
        g_drain(b)
        w_copy(b, (ng - 1) * _NB + b).start()
    for b in range(_NB):
        w_copy(b, (ng - 1) * _NB + b).wait()


def _sc_gather_news(news_pairs, hist_idx_w, news_idx):
    bh = hist_idx_w.shape[0]
    n_h = bh // (_NW * _CH2)
    n_b = _B // (_NW * _CH2)
    mesh = plsc.VectorSubcoreMesh(core_axis_name="c", subcore_axis_name="s")
    out_types = (
        jax.ShapeDtypeStruct((bh, _DP), jnp.float32),
        jax.ShapeDtypeStruct((_B, _DP), jnp.float32),
    )
    scratch = (
        [pltpu.VMEM((n_h * _CH2,), jnp.int32),
         pltpu.VMEM((n_b * _CH2,), jnp.int32)]
        + [pltpu.VMEM((_CH2, _DP), jnp.float32) for _ in range(_NB)]
        + [pltpu.SemaphoreType.DMA for _ in range(2 * _NB)]
    )

    @functools.partial(pl.kernel, mesh=mesh, out_type=out_types,
                       scratch_types=scratch)
    def k(news_t, hidx, nidx, out_h, out_n, hidx_v, nidx_v, *bufs_sems):
        bufs = bufs_sems[:_NB]
        gsem = bufs_sems[_NB:2 * _NB]
        wsem = bufs_sems[2 * _NB:]
        wid = lax.axis_index("s") * 2 + lax.axis_index("c")
        pltpu.sync_copy(hidx.at[pl.ds(wid * n_h * _CH2, n_h * _CH2)], hidx_v)
        pltpu.sync_copy(nidx.at[pl.ds(wid * n_b * _CH2, n_b * _CH2)], nidx_v)
        _sc_pipe(wid, news_t, hidx_v, out_h, n_h, bufs, gsem, wsem, True)
        _sc_pipe(wid, news_t, nidx_v, out_n, n_b, bufs, gsem, wsem, False)

    return k(news_pairs, hist_idx_w, news_idx)


def _sc_gather_user(user_pairs, user_idx, dep):
    n_b = _B // (_NW * _CH2)
    mesh = plsc.VectorSubcoreMesh(core_axis_name="c", subcore_axis_name="s")
    scratch = (
        [pltpu.VMEM((n_b * _CH2,), jnp.int32)]
        + [pltpu.VMEM((_CH2, _DP), jnp.float32) for _ in range(_NB)]
        + [pltpu.SemaphoreType.DMA for _ in range(2 * _NB)]
    )

    @functools.partial(
        pl.kernel, mesh=mesh,
        out_type=jax.ShapeDtypeStruct((_B, _DP), jnp.float32),
        scratch_types=scratch)
    def k(user_t, uidx, dep_r, out_u, uidx_v, *bufs_sems):
        del dep_r
        bufs = bufs_sems[:_NB]
        gsem = bufs_sems[_NB:2 * _NB]
        wsem = bufs_sems[2 * _NB:]
        wid = lax.axis_index("s") * 2 + lax.axis_index("c")
        pltpu.sync_copy(uidx.at[pl.ds(wid * n_b * _CH2, n_b * _CH2)], uidx_v)
        _sc_pipe(wid, user_t, uidx_v, out_u, n_b, bufs, gsem, wsem, False)

    return k(user_pairs, user_idx, dep)


def _half(pair, idx_col):
    odd = (idx_col & 1) == 1
    return jnp.where(odd, pair[:, _D:], pair[:, :_D])


def _tc_body(hist_ref, hidx_ref, msk_ref, negc_ref, upair_ref, uid_ref,
             npair_ref, nid_ref, w1p_ref, b1p_ref, w22_ref, pfold_ref,
             pdup_ref, r2_ref, b2_ref, wu_ref, bu_ref, wn_ref, bn_ref,
             out_ref):
    msk = msk_ref[...]
    a_parts = []
    for l in range(_LP):
        x = hist_ref[l].astype(jnp.float32)
        h = jnp.tanh(
            jnp.dot(x, w1p_ref[...], preferred_element_type=jnp.float32)
            + b1p_ref[...])
        a_parts.append(
            jnp.dot(h, w22_ref[...], preferred_element_type=jnp.float32))
    a104 = jnp.concatenate(a_parts, axis=1)
    a52 = jnp.dot(a104 * msk, pfold_ref[...],
                  preferred_element_type=jnp.float32) + b2_ref[...]
    a52 = jnp.where(hidx_ref[...] != 0, a52, negc_ref[...])
    m = jnp.max(a52, axis=1, keepdims=True)
    e = jnp.exp(a52 - m)
    s = jnp.sum(e, axis=1, keepdims=True)
    w52 = e / s
    w104 = jnp.dot(w52, pdup_ref[...],
                   preferred_element_type=jnp.float32)
    wexp = jnp.dot(w104 * msk, r2_ref[...],
                   preferred_element_type=jnp.float32)
    acc = jnp.zeros((_TB, _DP), jnp.float32)
    for l in range(_LP):
        acc = acc + hist_ref[l].astype(jnp.float32) * wexp[:, l * _DP:(l + 1) * _DP]
    hist_repr = acc[:, :_D] + acc[:, _D:]
    uemb = _half(upair_ref[...].astype(jnp.float32), uid_ref[...])
    nemb = _half(npair_ref[...].astype(jnp.float32), nid_ref[...])
    u = uemb + hist_repr
    ur = jnp.maximum(
        jnp.dot(u, wu_ref[...], preferred_element_type=jnp.float32)
        + bu_ref[...], 0.0)
    nr = jnp.maximum(
        jnp.dot(nemb, wn_ref[...], preferred_element_type=jnp.float32)
        + bn_ref[...], 0.0)
    out_ref[...] = jax.nn.sigmoid(jnp.sum(ur * nr, axis=1, keepdims=True))


def _tc_call(hist3, history_p, msk104, negc, gath_u, user_idx, gath_n,
             news_idx, w1p, b1p, w22, pfold, pdup, r2, b2r,
             W_user, b_user, W_news, b_news):
    grid = _B // _TB
    return pl.pallas_call(
        _tc_body,
        grid=(grid,),
        in_specs=[
            pl.BlockSpec((_LP, _TB, _DP), lambda i: (0, i, 0)),
            pl.BlockSpec((_TB, _LP), lambda i: (i, 0)),
            pl.BlockSpec((_TB, 2 * _LP), lambda i: (i, 0)),
            pl.BlockSpec((1, _LP), lambda i: (0, 0)),
            pl.BlockSpec((_TB, _DP), lambda i: (i, 0)),
            pl.BlockSpec((_TB, 1), lambda i: (i, 0)),
            pl.BlockSpec((_TB, _DP), lambda i: (i, 0)),
            pl.BlockSpec((_TB, 1), lambda i: (i, 0)),
            pl.BlockSpec((_DP, _D), lambda i: (0, 0)),
            pl.BlockSpec((1, _D), lambda i: (0, 0)),
            pl.BlockSpec((_D, 2), lambda i: (0, 0)),
            pl.BlockSpec((2 * _LP, _LP), lambda i: (0, 0)),
            pl.BlockSpec((_LP, 2 * _LP), lambda i: (0, 0)),
            pl.BlockSpec((2 * _LP, _LP * _DP), lambda i: (0, 0)),
            pl.BlockSpec((1, 1), lambda i: (0, 0)),
            pl.BlockSpec((_D, _D), lambda i: (0, 0)),
            pl.BlockSpec((1, _D), lambda i: (0, 0)),
            pl.BlockSpec((_D, _D), lambda i: (0, 0)),
            pl.BlockSpec((1, _D), lambda i: (0, 0)),
        ],
        out_specs=pl.BlockSpec((_TB, 1), lambda i: (i, 0)),
        out_shape=jax.ShapeDtypeStruct((_B, 1), jnp.float32),
    )(hist3, history_p, msk104, negc, gath_u, user_idx.reshape(_B, 1),
      gath_n, news_idx.reshape(_B, 1), w1p, b1p, w22, pfold, pdup, r2,
      b2r, W_user, b_user[None, :], W_news, b_news[None, :])


def kernel(user_idx, news_idx, history, user_table, news_table, W_user,
           b_user, W_news, b_news, W_a1, b_a1, W_a2, b_a2):
    history_p = jnp.concatenate(
        [history, jnp.zeros((_B, _LP - _L), history.dtype)], axis=1)
    hist_pair_idx = (history_p >> 1).T.reshape(-1)

    news_pairs = news_table.reshape(news_table.shape[0] // 2, _DP)
    user_pairs = user_table.reshape(user_table.shape[0] // 2, _DP)

    nch = hist_pair_idx.shape[0] // _CH2
    hist_idx_w = (hist_pair_idx.reshape(nch // _NW, _NW, _CH2)
                  .transpose(1, 0, 2).reshape(-1))

    gath_h, gath_n = _sc_gather_news(news_pairs, hist_idx_w, news_idx >> 1)
    gath_u = _sc_gather_user(user_pairs, user_idx >> 1, gath_n)
    hist3 = gath_h.reshape(_LP, _B, _DP)

    par = (history_p & 1).astype(jnp.float32)
    msk104 = jnp.stack([1.0 - par, par], axis=-1).reshape(_B, 2 * _LP)
    negc = jnp.concatenate(
        [jnp.full((1, _L), -1e9, jnp.float32),
         jnp.full((1, _LP - _L), -jnp.inf, jnp.float32)], axis=1)

    eye2 = jnp.eye(2, dtype=jnp.float32)
    w1p = jnp.kron(eye2, W_a1)
    b1p = jnp.tile(b_a1, 2)[None, :]
    w22 = jnp.kron(eye2, W_a2)
    eye52 = jnp.eye(_LP, dtype=jnp.float32)
    pfold = jnp.kron(eye52, jnp.ones((2, 1), jnp.float32))
    pdup = jnp.kron(eye52, jnp.ones((1, 2), jnp.float32))
    sel = jnp.zeros((2, _DP), jnp.float32)
    sel = sel.at[0, :_D].set(1.0).at[1, _D:].set(1.0)
    r2 = jnp.kron(eye52, sel)
    b2r = b_a2.reshape(1, 1)

    out = _tc_call(hist3, history_p, msk104, negc, gath_u, user_idx,
                   gath_n, news_idx, w1p, b1p, w22, pfold, pdup, r2, b2r,
                   W_user, b_user, W_news, b_news)
    return out[:, 0]
